# Initial kernel scaffold; baseline (speedup 1.0000x reference)
import jax
import jax.numpy as jnp
from jax.experimental import pallas as pl


def kernel(x, edge_index, pred_connectivity, W1a, b1a, W1b, b1b, W2a, b2a, W2b, b2b, Wp, bp):
    raise NotImplementedError("write your pallas kernel here")


# speedup vs baseline: 2.0666x; 2.0666x over previous
import functools

import jax
import jax.numpy as jnp
from jax import lax
from jax.experimental import pallas as pl
from jax.experimental.pallas import tpu as pltpu
from jax.experimental.pallas import tpu_sc as plsc

N = 19
E = 342
D_IN = 1025
HID = 256
OUT = 512

LANES = 16
E_PAD = ((E + LANES - 1) // LANES) * LANES
A_PAD = ((N * N + LANES - 1) // LANES) * LANES


def _sc_build_adjacency(src_hbm, dst_hbm, w_hbm, a_hbm, src_v, dst_v, w_v, a_v):
    cid = lax.axis_index("c")
    sid = lax.axis_index("s")

    @pl.when(jnp.logical_and(cid == 0, sid == 0))
    def _():
        pltpu.sync_copy(src_hbm, src_v)
        pltpu.sync_copy(dst_hbm, dst_v)
        pltpu.sync_copy(w_hbm, w_v)
        zero = jnp.zeros((LANES,), jnp.float32)
        for i in range(A_PAD // LANES):
            a_v[pl.ds(i * LANES, LANES)] = zero
        for e in range(E_PAD // LANES):
            s = src_v[pl.ds(e * LANES, LANES)]
            d = dst_v[pl.ds(e * LANES, LANES)]
            w = w_v[pl.ds(e * LANES, LANES)]
            idx = d * N + s
            plsc.addupdate_scatter(a_v, [idx], w)
        pltpu.sync_copy(a_v, a_hbm)


def _sc_adjacency_call(src, dst, w):
    run = functools.partial(
        pl.kernel,
        out_type=jax.ShapeDtypeStruct((A_PAD,), jnp.float32),
        mesh=plsc.VectorSubcoreMesh(core_axis_name="c", subcore_axis_name="s"),
        scratch_types=[
            pltpu.VMEM((E_PAD,), jnp.int32),
            pltpu.VMEM((E_PAD,), jnp.int32),
            pltpu.VMEM((E_PAD,), jnp.float32),
            pltpu.VMEM((A_PAD,), jnp.float32),
        ],
        compiler_params=pltpu.CompilerParams(needs_layout_passes=False),
    )(_sc_build_adjacency)
    return run(src, dst, w)


def _tc_dense(a_ref, x_ref, w1a_ref, b1a_ref, w1b_ref, b1b_ref,
              w2a_ref, b2a_ref, w2b_ref, b2b_ref, wp_ref, bp_ref, out_ref):
    a = a_ref[...]
    x = x_ref[...]
    z1 = x + jnp.dot(a, x, preferred_element_type=jnp.float32)
    t = jnp.maximum(
        jnp.dot(z1, w1a_ref[...], preferred_element_type=jnp.float32)
        + b1a_ref[...], 0.0)
    h1 = jnp.maximum(
        jnp.dot(t, w1b_ref[...], preferred_element_type=jnp.float32)
        + b1b_ref[...], 0.0)
    z2 = h1 + jnp.dot(a, h1, preferred_element_type=jnp.float32)
    u = jnp.maximum(
        jnp.dot(z2, w2a_ref[...], preferred_element_type=jnp.float32)
        + b2a_ref[...], 0.0)
    h2 = (jnp.dot(u, w2b_ref[...], preferred_element_type=jnp.float32)
          + b2b_ref[...])
    s = jnp.sum(h2 * wp_ref[...], keepdims=True) + bp_ref[...]
    out_ref[...] = 1.0 / (1.0 + jnp.exp(-s))


@jax.jit
def kernel(x, edge_index, pred_connectivity,
           W1a, b1a, W1b, b1b, W2a, b2a, W2b, b2b, Wp, bp):
    src = jnp.zeros((E_PAD,), jnp.int32).at[:E].set(edge_index[0])
    dst = jnp.zeros((E_PAD,), jnp.int32).at[:E].set(edge_index[1])
    w = jnp.zeros((E_PAD,), jnp.float32).at[:E].set(pred_connectivity)

    a_flat = _sc_adjacency_call(src, dst, w)
    a = a_flat[: N * N].reshape(N, N)

    out = pl.pallas_call(
        _tc_dense,
        out_shape=jax.ShapeDtypeStruct((1, 1), jnp.float32),
    )(
        a, x,
        W1a, b1a.reshape(1, HID), W1b, b1b.reshape(1, HID),
        W2a, b2a.reshape(1, OUT), W2b, b2b.reshape(1, OUT),
        Wp.reshape(N, OUT), bp.reshape(1, 1),
    )
    return out.reshape(1)


# speedup vs baseline: 2.3064x; 1.1160x over previous
import functools

import jax
import jax.numpy as jnp
from jax import lax
from jax.experimental import pallas as pl
from jax.experimental.pallas import tpu as pltpu
from jax.experimental.pallas import tpu_sc as plsc

N = 19
E = 342
D_IN = 1025
HID = 256
OUT = 512

LANES = 16
NCHUNK = (E + LANES - 1) // LANES
E_PAD = NCHUNK * LANES
TAIL = E - (NCHUNK - 1) * LANES
A_COLS = 32


def _sc_build_adjacency(src_hbm, dst_hbm, w_hbm, a_hbm, src_v, dst_v, w_v,
                        a_v, sem):
    cid = lax.axis_index("c")
    sid = lax.axis_index("s")

    @pl.when(jnp.logical_and(cid == 0, sid == 0))
    def _():
        c1 = pltpu.make_async_copy(src_hbm, src_v.at[pl.ds(0, E)], sem)
        c2 = pltpu.make_async_copy(dst_hbm, dst_v.at[pl.ds(0, E)], sem)
        c3 = pltpu.make_async_copy(w_hbm, w_v.at[pl.ds(0, E)], sem)
        c1.start()
        c2.start()
        c3.start()
        zero = jnp.zeros((LANES,), jnp.float32)
        for d in range(N):
            a_v[d, pl.ds(0, LANES)] = zero
            a_v[d, pl.ds(LANES, LANES)] = zero
        c1.wait()
        c2.wait()
        c3.wait()
        lane = lax.iota(jnp.int32, LANES)
        for e in range(NCHUNK):
            s = src_v[pl.ds(e * LANES, LANES)]
            d = dst_v[pl.ds(e * LANES, LANES)]
            w = w_v[pl.ds(e * LANES, LANES)]
            if e == NCHUNK - 1:
                plsc.addupdate_scatter(a_v, [d, s], w, mask=lane < TAIL)
            else:
                plsc.addupdate_scatter(a_v, [d, s], w)
        pltpu.sync_copy(a_v, a_hbm)


def _sc_adjacency_call(src, dst, w):
    run = functools.partial(
        pl.kernel,
        out_type=jax.ShapeDtypeStruct((N, A_COLS), jnp.float32),
        mesh=plsc.VectorSubcoreMesh(core_axis_name="c", subcore_axis_name="s"),
        scratch_types=[
            pltpu.VMEM((E_PAD,), jnp.int32),
            pltpu.VMEM((E_PAD,), jnp.int32),
            pltpu.VMEM((E_PAD,), jnp.float32),
            pltpu.VMEM((N, A_COLS), jnp.float32),
            pltpu.SemaphoreType.DMA,
        ],
        compiler_params=pltpu.CompilerParams(needs_layout_passes=False),
    )(_sc_build_adjacency)
    return run(src, dst, w)


def _tc_dense(a_ref, x_ref, w1a_ref, b1a_ref, w1b_ref, b1b_ref,
              w2a_ref, b2a_ref, w2b_ref, b2b_ref, wp_ref, bp_ref, out_ref):
    a = lax.slice(a_ref[...], (0, 0), (N, N))
    x = x_ref[...]
    z1 = x + jnp.dot(a, x, preferred_element_type=jnp.float32)
    t = jnp.maximum(
        jnp.dot(z1, w1a_ref[...], preferred_element_type=jnp.float32)
        + b1a_ref[...], 0.0)
    h1 = jnp.maximum(
        jnp.dot(t, w1b_ref[...], preferred_element_type=jnp.float32)
        + b1b_ref[...], 0.0)
    z2 = h1 + jnp.dot(a, h1, preferred_element_type=jnp.float32)
    u = jnp.maximum(
        jnp.dot(z2, w2a_ref[...], preferred_element_type=jnp.float32)
        + b2a_ref[...], 0.0)
    h2 = (jnp.dot(u, w2b_ref[...], preferred_element_type=jnp.float32)
          + b2b_ref[...])
    s = jnp.sum(h2 * wp_ref[...], keepdims=True) + bp_ref[...]
    out_ref[...] = 1.0 / (1.0 + jnp.exp(-s))


@jax.jit
def kernel(x, edge_index, pred_connectivity,
           W1a, b1a, W1b, b1b, W2a, b2a, W2b, b2b, Wp, bp):
    a = _sc_adjacency_call(edge_index[0], edge_index[1], pred_connectivity)

    out = pl.pallas_call(
        _tc_dense,
        out_shape=jax.ShapeDtypeStruct((1, 1), jnp.float32),
    )(
        a, x,
        W1a, b1a.reshape(1, HID), W1b, b1b.reshape(1, HID),
        W2a, b2a.reshape(1, OUT), W2b, b2b.reshape(1, OUT),
        Wp.reshape(N, OUT), bp.reshape(1, 1),
    )
    return out.reshape(1)
